# baseline (speedup 1.0000x reference)
import functools

import jax
import jax.numpy as jnp
from jax import lax
from jax.experimental import pallas as pl
from jax.experimental.pallas import tpu as pltpu
from jax.experimental.pallas import tpu_sc as plsc

BATCH = 16384
K = 128
NW = 32
BPW = BATCH // NW
NCHUNK = 4
CHUNK = BPW // NCHUNK
GROUPS = CHUNK // 16
UNR = 16

_MESH = plsc.VectorSubcoreMesh(core_axis_name="c", subcore_axis_name="s")
_PARAMS = pltpu.CompilerParams(needs_layout_passes=False)


def _dots_body(user_hbm, item_hbm, gu_hbm, gi_hbm, mu_hbm, out_hbm,
               idx_u, idx_i, ru0, ru1, ri0, ri1, mu_v, out_v, sem0, sem1):
    c = lax.axis_index("c")
    s = lax.axis_index("s")
    wid = s * 2 + c
    base = wid * BPW

    pltpu.sync_copy(mu_hbm, mu_v)
    for ch in range(NCHUNK):
        pltpu.sync_copy(user_hbm.at[pl.ds(base + ch * CHUNK, CHUNK)],
                        idx_u.at[ch])
        pltpu.sync_copy(item_hbm.at[pl.ds(base + ch * CHUNK, CHUNK)],
                        idx_i.at[ch])
    mu = mu_v[...]

    rbufs = [(ru0, ri0, sem0), (ru1, ri1, sem1)]

    def fire(ch):
        ru, ri, sem = rbufs[ch % 2]
        return (pltpu.async_copy(gu_hbm.at[idx_u.at[ch]], ru, sem),
                pltpu.async_copy(gi_hbm.at[idx_i.at[ch]], ri, sem))

    pending = fire(0)
    for ch in range(NCHUNK):
        nxt = fire(ch + 1) if ch + 1 < NCHUNK else None
        pending[0].wait()
        pending[1].wait()
        ru, ri, _ = rbufs[ch % 2]

        def group_body(g, carry, ru=ru, ri=ri, ch=ch):
            lane = lax.iota(jnp.int32, 16)
            rid = g * 16 + lane

            def k_body(j, accs, rid=rid, ru=ru, ri=ri):
                accs = list(accs)
                cb = rid + j * UNR
                for t in range(UNR):
                    col = (cb + t) & (K - 1)
                    uu = plsc.load_gather(ru, [rid, col])
                    vv = plsc.load_gather(ri, [rid, col])
                    accs[t % 4] = accs[t % 4] + uu * vv
                return tuple(accs)

            z = jnp.zeros((16,), jnp.float32)
            accs = lax.fori_loop(0, K // UNR, k_body, (z, z, z, z))
            acc = (accs[0] + accs[1]) + (accs[2] + accs[3])
            out_v[pl.ds(ch * CHUNK + g * 16, 16)] = acc + mu
            return carry

        lax.fori_loop(0, GROUPS, group_body, 0)
        pending = nxt

    pltpu.sync_copy(out_v, out_hbm.at[pl.ds(base, BPW)])


_dots_sc = functools.partial(
    pl.kernel,
    out_type=jax.ShapeDtypeStruct((BATCH,), jnp.float32),
    mesh=_MESH,
    compiler_params=_PARAMS,
    scratch_types=[
        pltpu.VMEM((NCHUNK, CHUNK), jnp.int32),
        pltpu.VMEM((NCHUNK, CHUNK), jnp.int32),
        pltpu.VMEM((CHUNK, K), jnp.float32),
        pltpu.VMEM((CHUNK, K), jnp.float32),
        pltpu.VMEM((CHUNK, K), jnp.float32),
        pltpu.VMEM((CHUNK, K), jnp.float32),
        pltpu.VMEM((16,), jnp.float32),
        pltpu.VMEM((BPW,), jnp.float32),
        pltpu.SemaphoreType.DMA,
        pltpu.SemaphoreType.DMA,
    ],
)(_dots_body)


def _bias_add_body(dots_hbm, bu_hbm, bi_hbm, out_hbm, d_v, bu_v, bi_v, sem):
    c = lax.axis_index("c")
    s = lax.axis_index("s")
    wid = s * 2 + c
    base = wid * BPW

    cps = [pltpu.async_copy(dots_hbm.at[pl.ds(base, BPW)], d_v, sem),
           pltpu.async_copy(bu_hbm.at[pl.ds(base, BPW)], bu_v, sem),
           pltpu.async_copy(bi_hbm.at[pl.ds(base, BPW)], bi_v, sem)]
    for cp in cps:
        cp.wait()

    def body(g, carry):
        sl = pl.ds(g * 16, 16)
        d_v[sl] = d_v[sl] + bu_v[sl] + bi_v[sl]
        return carry

    lax.fori_loop(0, BPW // 16, body, 0)
    pltpu.sync_copy(d_v, out_hbm.at[pl.ds(base, BPW)])


_bias_add_sc = functools.partial(
    pl.kernel,
    out_type=jax.ShapeDtypeStruct((BATCH,), jnp.float32),
    mesh=_MESH,
    compiler_params=_PARAMS,
    scratch_types=[
        pltpu.VMEM((BPW,), jnp.float32),
        pltpu.VMEM((BPW,), jnp.float32),
        pltpu.VMEM((BPW,), jnp.float32),
        pltpu.SemaphoreType.DMA,
    ],
)(_bias_add_body)


def kernel(user, item, Gu, Gi, Bu, Bi, Mu):
    ui = user.astype(jnp.int32)
    ii = item.astype(jnp.int32)
    mu16 = jnp.broadcast_to(jnp.reshape(Mu, (1,)), (16,))
    dots = _dots_sc(ui, ii, Gu, Gi, mu16)
    bu_g = jnp.take(Bu, ui, axis=0)[:, 0]
    bi_g = jnp.take(Bi, ii, axis=0)[:, 0]
    return _bias_add_sc(dots, bu_g, bi_g)


# speedup vs baseline: 1.4002x; 1.4002x over previous
import functools

import jax
import jax.numpy as jnp
from jax import lax
from jax.experimental import pallas as pl
from jax.experimental.pallas import tpu as pltpu
from jax.experimental.pallas import tpu_sc as plsc

BATCH = 16384
K = 128
NW = 32
BPW = BATCH // NW
NCHUNK = 4
CHUNK = BPW // NCHUNK
GROUPS = CHUNK // 16
UNR = 16

_MESH = plsc.VectorSubcoreMesh(core_axis_name="c", subcore_axis_name="s")
_PARAMS = pltpu.CompilerParams(needs_layout_passes=False)


def _dots_body(user_hbm, item_hbm, gu_hbm, gi_hbm, mu_hbm, out_hbm,
               idx_u, idx_i, ru0, ru1, ri0, ri1, mu_v, out_v, sem0, sem1):
    c = lax.axis_index("c")
    s = lax.axis_index("s")
    wid = s * 2 + c
    base = wid * BPW

    pltpu.sync_copy(mu_hbm, mu_v)
    for ch in range(NCHUNK):
        pltpu.sync_copy(user_hbm.at[pl.ds(base + ch * CHUNK, CHUNK)],
                        idx_u.at[ch])
        pltpu.sync_copy(item_hbm.at[pl.ds(base + ch * CHUNK, CHUNK)],
                        idx_i.at[ch])
    mu = mu_v[...]

    rbufs = [(ru0, ri0, sem0), (ru1, ri1, sem1)]

    def fire(ch):
        ru, ri, sem = rbufs[ch % 2]
        return (pltpu.async_copy(gu_hbm.at[idx_u.at[ch]], ru, sem),
                pltpu.async_copy(gi_hbm.at[idx_i.at[ch]], ri, sem))

    pending = fire(0)
    for ch in range(NCHUNK):
        nxt = fire(ch + 1) if ch + 1 < NCHUNK else None
        pending[0].wait()
        pending[1].wait()
        ru, ri, _ = rbufs[ch % 2]

        def group_body(g, carry, ru=ru, ri=ri, ch=ch):
            lane = lax.iota(jnp.int32, 16)
            rid = g * 16 + lane

            def k_body(j, accs, rid=rid, ru=ru, ri=ri):
                accs = list(accs)
                cb = rid + j * UNR
                for t in range(UNR):
                    col = (cb + t) & (K - 1)
                    uu = plsc.load_gather(ru, [rid, col])
                    vv = plsc.load_gather(ri, [rid, col])
                    accs[t % 4] = accs[t % 4] + uu * vv
                return tuple(accs)

            z = jnp.zeros((16,), jnp.float32)
            accs = lax.fori_loop(0, K // UNR, k_body, (z, z, z, z))
            acc = (accs[0] + accs[1]) + (accs[2] + accs[3])
            out_v[pl.ds(ch * CHUNK + g * 16, 16)] = acc + mu
            return carry

        lax.fori_loop(0, GROUPS, group_body, 0)
        pending = nxt

    pltpu.sync_copy(out_v, out_hbm.at[pl.ds(base, BPW)])


_dots_sc = functools.partial(
    pl.kernel,
    out_type=jax.ShapeDtypeStruct((BATCH,), jnp.float32),
    mesh=_MESH,
    compiler_params=_PARAMS,
    scratch_types=[
        pltpu.VMEM((NCHUNK, CHUNK), jnp.int32),
        pltpu.VMEM((NCHUNK, CHUNK), jnp.int32),
        pltpu.VMEM((CHUNK, K), jnp.float32),
        pltpu.VMEM((CHUNK, K), jnp.float32),
        pltpu.VMEM((CHUNK, K), jnp.float32),
        pltpu.VMEM((CHUNK, K), jnp.float32),
        pltpu.VMEM((16,), jnp.float32),
        pltpu.VMEM((BPW,), jnp.float32),
        pltpu.SemaphoreType.DMA,
        pltpu.SemaphoreType.DMA,
    ],
)(_dots_body)


def _bias_add_body(user_hbm, item_hbm, dots_hbm, bu_hbm, bi_hbm, out_hbm,
                   idx_u, idx_i, d_v, bu_v, bi_v, sem, semb):
    c = lax.axis_index("c")
    s = lax.axis_index("s")
    wid = s * 2 + c
    base = wid * BPW

    for ch in range(NCHUNK):
        pltpu.sync_copy(user_hbm.at[pl.ds(base + ch * CHUNK, CHUNK)],
                        idx_u.at[ch])
        pltpu.sync_copy(item_hbm.at[pl.ds(base + ch * CHUNK, CHUNK)],
                        idx_i.at[ch])
    cps = [pltpu.async_copy(dots_hbm.at[pl.ds(base, BPW)], d_v, sem)]
    for ch in range(NCHUNK):
        cps.append(pltpu.async_copy(
            bu_hbm.at[idx_u.at[ch]], bu_v.at[pl.ds(ch * CHUNK, CHUNK)], semb))
        cps.append(pltpu.async_copy(
            bi_hbm.at[idx_i.at[ch]], bi_v.at[pl.ds(ch * CHUNK, CHUNK)], semb))
    for cp in cps:
        cp.wait()

    def body(g, carry):
        sl = pl.ds(g * 16, 16)
        d_v[sl] = d_v[sl] + bu_v[sl] + bi_v[sl]
        return carry

    lax.fori_loop(0, BPW // 16, body, 0)
    pltpu.sync_copy(d_v, out_hbm.at[pl.ds(base, BPW)])


_bias_add_sc = functools.partial(
    pl.kernel,
    out_type=jax.ShapeDtypeStruct((BATCH,), jnp.float32),
    mesh=_MESH,
    compiler_params=_PARAMS,
    scratch_types=[
        pltpu.VMEM((NCHUNK, CHUNK), jnp.int32),
        pltpu.VMEM((NCHUNK, CHUNK), jnp.int32),
        pltpu.VMEM((BPW,), jnp.float32),
        pltpu.VMEM((BPW,), jnp.float32),
        pltpu.VMEM((BPW,), jnp.float32),
        pltpu.SemaphoreType.DMA,
        pltpu.SemaphoreType.DMA,
    ],
)(_bias_add_body)


def kernel(user, item, Gu, Gi, Bu, Bi, Mu):
    ui = user.astype(jnp.int32)
    ii = item.astype(jnp.int32)
    mu16 = jnp.broadcast_to(jnp.reshape(Mu, (1,)), (16,))
    dots = _dots_sc(ui, ii, Gu, Gi, mu16)
    bu1d = jnp.squeeze(Bu, -1)
    bi1d = jnp.squeeze(Bi, -1)
    bu1d, bi1d, dots_b = jax.lax.optimization_barrier((bu1d, bi1d, dots))
    return _bias_add_sc(ui, ii, dots_b, bu1d, bi1d)


# speedup vs baseline: 1.4622x; 1.0443x over previous
import functools

import jax
import jax.numpy as jnp
from jax import lax
from jax.experimental import pallas as pl
from jax.experimental.pallas import tpu as pltpu
from jax.experimental.pallas import tpu_sc as plsc

BATCH = 16384
K = 128
NW = 32
BPW = BATCH // NW
NCHUNK = 4
CHUNK = BPW // NCHUNK
GROUPS = CHUNK // 16
UNR = 16

_MESH = plsc.VectorSubcoreMesh(core_axis_name="c", subcore_axis_name="s")
_PARAMS = pltpu.CompilerParams(needs_layout_passes=False)


def _dots_body(user_hbm, item_hbm, gu_hbm, gi_hbm, mu_hbm, out_hbm,
               idx_u, idx_i, ru0, ru1, ri0, ri1, mu_v, out_v, sem0, sem1):
    c = lax.axis_index("c")
    s = lax.axis_index("s")
    wid = s * 2 + c
    base = wid * BPW

    pltpu.sync_copy(mu_hbm, mu_v)
    for ch in range(NCHUNK):
        pltpu.sync_copy(user_hbm.at[pl.ds(base + ch * CHUNK, CHUNK)],
                        idx_u.at[ch])
        pltpu.sync_copy(item_hbm.at[pl.ds(base + ch * CHUNK, CHUNK)],
                        idx_i.at[ch])
    mu = mu_v[...]

    rbufs = [(ru0, ri0, sem0), (ru1, ri1, sem1)]

    def fire(ch):
        ru, ri, sem = rbufs[ch % 2]
        return (pltpu.async_copy(gu_hbm.at[idx_u.at[ch]], ru, sem),
                pltpu.async_copy(gi_hbm.at[idx_i.at[ch]], ri, sem))

    pending = fire(0)
    for ch in range(NCHUNK):
        nxt = fire(ch + 1) if ch + 1 < NCHUNK else None
        pending[0].wait()
        pending[1].wait()
        ru, ri, _ = rbufs[ch % 2]

        def group_body(g, carry, ru=ru, ri=ri, ch=ch):
            lane = lax.iota(jnp.int32, 16)
            rid = g * 16 + lane

            def k_body(j, accs, rid=rid, ru=ru, ri=ri):
                accs = list(accs)
                cb = rid + j * UNR
                for t in range(UNR):
                    col = (cb + t) & (K - 1)
                    uu = plsc.load_gather(ru, [rid, col])
                    vv = plsc.load_gather(ri, [rid, col])
                    accs[t % 4] = accs[t % 4] + uu * vv
                return tuple(accs)

            z = jnp.zeros((16,), jnp.float32)
            accs = lax.fori_loop(0, K // UNR, k_body, (z, z, z, z))
            acc = (accs[0] + accs[1]) + (accs[2] + accs[3])
            out_v[pl.ds(ch * CHUNK + g * 16, 16)] = acc + mu
            return carry

        lax.fori_loop(0, GROUPS, group_body, 0)
        pending = nxt

    pltpu.sync_copy(out_v, out_hbm.at[pl.ds(base, BPW)])


_dots_sc = functools.partial(
    pl.kernel,
    out_type=jax.ShapeDtypeStruct((BATCH,), jnp.float32),
    mesh=_MESH,
    compiler_params=_PARAMS,
    scratch_types=[
        pltpu.VMEM((NCHUNK, CHUNK), jnp.int32),
        pltpu.VMEM((NCHUNK, CHUNK), jnp.int32),
        pltpu.VMEM((CHUNK, K), jnp.float32),
        pltpu.VMEM((CHUNK, K), jnp.float32),
        pltpu.VMEM((CHUNK, K), jnp.float32),
        pltpu.VMEM((CHUNK, K), jnp.float32),
        pltpu.VMEM((16,), jnp.float32),
        pltpu.VMEM((BPW,), jnp.float32),
        pltpu.SemaphoreType.DMA,
        pltpu.SemaphoreType.DMA,
    ],
)(_dots_body)


def _bias_add_body(user_hbm, item_hbm, dots_hbm, bu_hbm, bi_hbm, out_hbm,
                   idx_u, idx_i, d_v, bu_v, bi_v, sem, semb):
    c = lax.axis_index("c")
    s = lax.axis_index("s")
    wid = s * 2 + c
    base = wid * BPW

    dcp = pltpu.async_copy(dots_hbm.at[pl.ds(base, BPW)], d_v, sem)
    icps = []
    for ch in range(NCHUNK):
        icps.append(pltpu.async_copy(
            user_hbm.at[pl.ds(base + ch * CHUNK, CHUNK)], idx_u.at[ch], sem))
        icps.append(pltpu.async_copy(
            item_hbm.at[pl.ds(base + ch * CHUNK, CHUNK)], idx_i.at[ch], sem))
    for cp in icps:
        cp.wait()
    cps = [dcp]
    for ch in range(NCHUNK):
        cps.append(pltpu.async_copy(
            bu_hbm.at[idx_u.at[ch]], bu_v.at[pl.ds(ch * CHUNK, CHUNK)], semb))
        cps.append(pltpu.async_copy(
            bi_hbm.at[idx_i.at[ch]], bi_v.at[pl.ds(ch * CHUNK, CHUNK)], semb))
    for cp in cps:
        cp.wait()

    def body(g, carry):
        sl = pl.ds(g * 16, 16)
        d_v[sl] = d_v[sl] + bu_v[sl] + bi_v[sl]
        return carry

    lax.fori_loop(0, BPW // 16, body, 0)
    pltpu.sync_copy(d_v, out_hbm.at[pl.ds(base, BPW)])


_bias_add_sc = functools.partial(
    pl.kernel,
    out_type=jax.ShapeDtypeStruct((BATCH,), jnp.float32),
    mesh=_MESH,
    compiler_params=_PARAMS,
    scratch_types=[
        pltpu.VMEM((NCHUNK, CHUNK), jnp.int32),
        pltpu.VMEM((NCHUNK, CHUNK), jnp.int32),
        pltpu.VMEM((BPW,), jnp.float32),
        pltpu.VMEM((BPW,), jnp.float32),
        pltpu.VMEM((BPW,), jnp.float32),
        pltpu.SemaphoreType.DMA,
        pltpu.SemaphoreType.DMA,
    ],
)(_bias_add_body)


def kernel(user, item, Gu, Gi, Bu, Bi, Mu):
    ui = user.astype(jnp.int32)
    ii = item.astype(jnp.int32)
    mu16 = jnp.broadcast_to(jnp.reshape(Mu, (1,)), (16,))
    dots = _dots_sc(ui, ii, Gu, Gi, mu16)
    bu1d = jnp.squeeze(Bu, -1)
    bi1d = jnp.squeeze(Bi, -1)
    bu1d, bi1d, dots_b = jax.lax.optimization_barrier((bu1d, bi1d, dots))
    return _bias_add_sc(ui, ii, dots_b, bu1d, bi1d)
